# baseline (speedup 1.0000x reference)
import functools

import jax
import jax.numpy as jnp
from jax import lax
from jax.experimental import pallas as pl
from jax.experimental.pallas import tpu as pltpu
from jax.experimental.pallas import tpu_sc as plsc

B = 2
N = 8192
M = 8192
K = 32
P = 15
P_PAD = 16
IN_C = 64
OUT_C = 64
SIGMA = 1.0
EPS = 1e-5

Q = B * N
ROWS = Q * K
NW = 32
RPW = ROWS // NW
CH = 128
NCH = RPW // CH

BLK = 128
NBLK = Q // BLK


def _sc_gather_body(f_hbm, s_hbm, nb_hbm, fnb_hbm, snb_hbm,
                    idx_v, frows, srows, semf, sems):
    wid = lax.axis_index("s") * 2 + lax.axis_index("c")
    base0 = wid * RPW

    def body(j, carry):
        base = base0 + j * CH
        pltpu.sync_copy(nb_hbm.at[pl.ds(base, CH)], idx_v)
        cf = pltpu.async_copy(f_hbm.at[idx_v], frows, semf)
        cs = pltpu.async_copy(s_hbm.at[idx_v], srows, sems)
        cf.wait()
        cs.wait()
        pltpu.sync_copy(frows, fnb_hbm.at[pl.ds(base, CH)])
        pltpu.sync_copy(srows, snb_hbm.at[pl.ds(base, CH)])
        return carry

    lax.fori_loop(0, NCH, body, 0)


def _make_sc_gather():
    return pl.kernel(
        _sc_gather_body,
        out_type=[jax.ShapeDtypeStruct((ROWS, IN_C), jnp.bfloat16),
                  jax.ShapeDtypeStruct((ROWS, 16), jnp.float32)],
        mesh=plsc.VectorSubcoreMesh(core_axis_name="c",
                                    subcore_axis_name="s"),
        scratch_types=[pltpu.VMEM((CH,), jnp.int32),
                       pltpu.VMEM((CH, IN_C), jnp.bfloat16),
                       pltpu.VMEM((CH, 16), jnp.float32),
                       pltpu.SemaphoreType.DMA,
                       pltpu.SemaphoreType.DMA],
        compiler_params=pltpu.CompilerParams(use_tc_tiling_on_sc=False),
    )


def _stage1_kernel(q_ref, snb_ref, fnb_ref, kpt_ref, kpn_ref, exp_ref,
                   w2_ref, bias_ref, conv_ref, sums_ref):
    i = pl.program_id(0)

    qrep = jnp.broadcast_to(q_ref[...][:, None, :],
                            (BLK, K, 16)).reshape(BLK * K, 16)
    d3 = snb_ref[...] - qrep

    cross = jnp.dot(d3, kpt_ref[...],
                    preferred_element_type=jnp.float32)
    nrm = jnp.sum(d3 * d3, axis=1, keepdims=True)
    sq = jnp.maximum(nrm + kpn_ref[0][None, :] - 2.0 * cross, 0.0)
    infl = jnp.maximum(1.0 - jnp.sqrt(sq) / SIGMA, 0.0)

    inflx = jnp.dot(infl, exp_ref[...],
                    preferred_element_type=jnp.float32)
    fnb = fnb_ref[...].astype(jnp.float32)
    fnbx = jnp.concatenate([fnb] * P_PAD, axis=1)
    wf = inflx * fnbx
    weighted = jnp.sum(wf.reshape(BLK, K, P_PAD * IN_C), axis=1)

    conv = jnp.dot(weighted, w2_ref[...],
                   preferred_element_type=jnp.float32)

    nsum = jnp.sum(fnb, axis=1)
    valid = (jnp.abs(nsum) > 0.0).astype(jnp.float32).reshape(BLK, K)
    ncount = jnp.maximum(jnp.sum(valid, axis=1), 1.0)
    conv = conv / ncount[:, None] + bias_ref[0][None, :]

    conv_ref[...] = conv

    rows = jnp.stack([jnp.sum(conv, axis=0),
                      jnp.sum(conv * conv, axis=0)], axis=0)
    rows = jnp.concatenate(
        [rows, jnp.zeros((2, 128 - OUT_C), jnp.float32)], axis=1)
    s = jnp.concatenate([rows, jnp.zeros((6, 128), jnp.float32)], axis=0)

    @pl.when(i == 0)
    def _():
        sums_ref[...] = s

    @pl.when(i != 0)
    def _():
        sums_ref[...] += s


def _stage2_kernel(conv_ref, sums_ref, gamma_ref, beta_ref, out_ref):
    mean = sums_ref[0, :OUT_C] / float(Q)
    var = sums_ref[1, :OUT_C] / float(Q) - mean * mean
    inv = lax.rsqrt(var + EPS)
    xn = (conv_ref[...] - mean[None, :]) * inv[None, :]
    xn = xn * gamma_ref[0][None, :] + beta_ref[0][None, :]
    out_ref[...] = jnp.maximum(xn, 0.0)


def kernel(query, support, features, neighbors, kernel_points, weights,
           bias, gamma, beta):
    nb1d = (neighbors.astype(jnp.int32)
            + jnp.arange(B, dtype=jnp.int32)[:, None, None] * M
            ).reshape(ROWS)
    f_flat = features.astype(jnp.bfloat16).reshape(B * M, IN_C)
    s_pad = jnp.concatenate(
        [support.reshape(B * M, 3),
         jnp.zeros((B * M, 13), jnp.float32)], axis=1)

    fnb, snb = _make_sc_gather()(f_flat, s_pad, nb1d)

    q16 = jnp.concatenate(
        [query.reshape(Q, 3), jnp.zeros((Q, 13), jnp.float32)], axis=1)

    kpt = jnp.zeros((16, 16), dtype=jnp.float32)
    kpt = kpt.at[:3, :P].set(kernel_points.T)
    kpn = jnp.zeros((8, 16), dtype=jnp.float32)
    kpn = kpn.at[0, :P].set(jnp.sum(kernel_points * kernel_points, axis=1))
    expand = jnp.zeros((16, P_PAD * IN_C), dtype=jnp.float32)
    for p in range(P):
        expand = expand.at[p, p * IN_C:(p + 1) * IN_C].set(1.0)

    w2 = jnp.concatenate(
        [weights.reshape(P * IN_C, OUT_C),
         jnp.zeros((IN_C, OUT_C), dtype=jnp.float32)], axis=0)

    bias2 = jnp.zeros((8, OUT_C), dtype=jnp.float32).at[0].set(bias)
    gamma2 = jnp.zeros((8, OUT_C), dtype=jnp.float32).at[0].set(gamma)
    beta2 = jnp.zeros((8, OUT_C), dtype=jnp.float32).at[0].set(beta)

    conv, sums = pl.pallas_call(
        _stage1_kernel,
        grid=(NBLK,),
        compiler_params=pltpu.CompilerParams(
            vmem_limit_bytes=110 * 1024 * 1024),
        in_specs=[
            pl.BlockSpec((BLK, 16), lambda i: (i, 0)),
            pl.BlockSpec((BLK * K, 16), lambda i: (i, 0)),
            pl.BlockSpec((BLK * K, IN_C), lambda i: (i, 0)),
            pl.BlockSpec((16, 16), lambda i: (0, 0)),
            pl.BlockSpec((8, 16), lambda i: (0, 0)),
            pl.BlockSpec((16, P_PAD * IN_C), lambda i: (0, 0)),
            pl.BlockSpec((P_PAD * IN_C, OUT_C), lambda i: (0, 0)),
            pl.BlockSpec((8, OUT_C), lambda i: (0, 0)),
        ],
        out_specs=[
            pl.BlockSpec((BLK, OUT_C), lambda i: (i, 0)),
            pl.BlockSpec((8, 128), lambda i: (0, 0)),
        ],
        out_shape=[
            jax.ShapeDtypeStruct((Q, OUT_C), jnp.float32),
            jax.ShapeDtypeStruct((8, 128), jnp.float32),
        ],
    )(q16, snb, fnb, kpt, kpn, expand, w2, bias2)

    out = pl.pallas_call(
        _stage2_kernel,
        grid=(NBLK,),
        in_specs=[
            pl.BlockSpec((BLK, OUT_C), lambda i: (i, 0)),
            pl.BlockSpec((8, 128), lambda i: (0, 0)),
            pl.BlockSpec((8, OUT_C), lambda i: (0, 0)),
            pl.BlockSpec((8, OUT_C), lambda i: (0, 0)),
        ],
        out_specs=pl.BlockSpec((BLK, OUT_C), lambda i: (i, 0)),
        out_shape=jax.ShapeDtypeStruct((Q, OUT_C), jnp.float32),
    )(conv, sums, gamma2, beta2)

    return out.reshape(B, N, OUT_C)


# speedup vs baseline: 1.1801x; 1.1801x over previous
import functools

import jax
import jax.numpy as jnp
from jax import lax
from jax.experimental import pallas as pl
from jax.experimental.pallas import tpu as pltpu
from jax.experimental.pallas import tpu_sc as plsc

B = 2
N = 8192
M = 8192
K = 32
P = 15
P_PAD = 16
IN_C = 64
OUT_C = 64
SIGMA = 1.0
EPS = 1e-5

Q = B * N
ROWS = Q * K
NW = 32
CHUNKS = 2
QC = Q // CHUNKS
ROWSC = QC * K
RPW = ROWSC // NW
CH = 128
NCH = RPW // CH

BLK = 128
NBLK = QC // BLK


def _sc_gather_body(f_hbm, s_hbm, nb_hbm, fnb_hbm, snb_hbm,
                    idx_v, frows, srows, semf, sems):
    wid = lax.axis_index("s") * 2 + lax.axis_index("c")
    base0 = wid * RPW

    def body(j, carry):
        base = base0 + j * CH
        pltpu.sync_copy(nb_hbm.at[pl.ds(base, CH)], idx_v)
        cf = pltpu.async_copy(f_hbm.at[idx_v], frows, semf)
        cs = pltpu.async_copy(s_hbm.at[idx_v], srows, sems)
        cf.wait()
        cs.wait()
        pltpu.sync_copy(frows, fnb_hbm.at[pl.ds(base, CH)])
        pltpu.sync_copy(srows, snb_hbm.at[pl.ds(base, CH)])
        return carry

    lax.fori_loop(0, NCH, body, 0)


def _make_sc_gather():
    return pl.kernel(
        _sc_gather_body,
        out_type=[jax.ShapeDtypeStruct((ROWSC, IN_C), jnp.float32),
                  jax.ShapeDtypeStruct((ROWSC, 16), jnp.float32)],
        mesh=plsc.VectorSubcoreMesh(core_axis_name="c",
                                    subcore_axis_name="s"),
        scratch_types=[pltpu.VMEM((CH,), jnp.int32),
                       pltpu.VMEM((CH, IN_C), jnp.float32),
                       pltpu.VMEM((CH, 16), jnp.float32),
                       pltpu.SemaphoreType.DMA,
                       pltpu.SemaphoreType.DMA],
        compiler_params=pltpu.CompilerParams(use_tc_tiling_on_sc=False),
    )


def _stage1_kernel(q_ref, snb_ref, fnb_ref, kpt_ref, kpn_ref, exp_ref,
                   w2_ref, bias_ref, conv_ref, sums_ref):
    i = pl.program_id(0)

    qrep = jnp.broadcast_to(q_ref[...][:, None, :],
                            (BLK, K, 16)).reshape(BLK * K, 16)
    d3 = snb_ref[...] - qrep

    cross = jnp.dot(d3, kpt_ref[...],
                    preferred_element_type=jnp.float32)
    nrm = jnp.sum(d3 * d3, axis=1, keepdims=True)
    sq = jnp.maximum(nrm + kpn_ref[0][None, :] - 2.0 * cross, 0.0)
    infl = jnp.maximum(1.0 - jnp.sqrt(sq) / SIGMA, 0.0)

    inflx = jnp.dot(infl, exp_ref[...],
                    preferred_element_type=jnp.float32)
    fnb = fnb_ref[...]
    fnbx = jnp.concatenate([fnb] * P_PAD, axis=1)
    wf = inflx * fnbx
    weighted = jnp.sum(wf.reshape(BLK, K, P_PAD * IN_C), axis=1)

    conv = jnp.dot(weighted, w2_ref[...],
                   preferred_element_type=jnp.float32)

    nsum = jnp.sum(fnb, axis=1)
    valid = (jnp.abs(nsum) > 0.0).astype(jnp.float32).reshape(BLK, K)
    ncount = jnp.maximum(jnp.sum(valid, axis=1), 1.0)
    conv = conv / ncount[:, None] + bias_ref[0][None, :]

    conv_ref[...] = conv

    rows = jnp.stack([jnp.sum(conv, axis=0),
                      jnp.sum(conv * conv, axis=0)], axis=0)
    rows = jnp.concatenate(
        [rows, jnp.zeros((2, 128 - OUT_C), jnp.float32)], axis=1)
    s = jnp.concatenate([rows, jnp.zeros((6, 128), jnp.float32)], axis=0)

    @pl.when(i == 0)
    def _():
        sums_ref[...] = s

    @pl.when(i != 0)
    def _():
        sums_ref[...] += s


def _stage2_kernel(conv_ref, sums_a_ref, sums_b_ref, gamma_ref, beta_ref,
                   out_ref):
    s0 = sums_a_ref[0, :OUT_C] + sums_b_ref[0, :OUT_C]
    s1 = sums_a_ref[1, :OUT_C] + sums_b_ref[1, :OUT_C]
    mean = s0 / float(Q)
    var = s1 / float(Q) - mean * mean
    inv = lax.rsqrt(var + EPS)
    xn = (conv_ref[...] - mean[None, :]) * inv[None, :]
    xn = xn * gamma_ref[0][None, :] + beta_ref[0][None, :]
    out_ref[...] = jnp.maximum(xn, 0.0)


def kernel(query, support, features, neighbors, kernel_points, weights,
           bias, gamma, beta):
    nb1d = (neighbors.astype(jnp.int32)
            + jnp.arange(B, dtype=jnp.int32)[:, None, None] * M
            ).reshape(ROWS)
    f_flat = features.reshape(B * M, IN_C)
    s_pad = jnp.concatenate(
        [support.reshape(B * M, 3),
         jnp.zeros((B * M, 13), jnp.float32)], axis=1)


    q16 = jnp.concatenate(
        [query.reshape(Q, 3), jnp.zeros((Q, 13), jnp.float32)], axis=1)

    kpt = jnp.zeros((16, 16), dtype=jnp.float32)
    kpt = kpt.at[:3, :P].set(kernel_points.T)
    kpn = jnp.zeros((8, 16), dtype=jnp.float32)
    kpn = kpn.at[0, :P].set(jnp.sum(kernel_points * kernel_points, axis=1))
    expand = jnp.zeros((16, P_PAD * IN_C), dtype=jnp.float32)
    for p in range(P):
        expand = expand.at[p, p * IN_C:(p + 1) * IN_C].set(1.0)

    w2 = jnp.concatenate(
        [weights.reshape(P * IN_C, OUT_C),
         jnp.zeros((IN_C, OUT_C), dtype=jnp.float32)], axis=0)

    bias2 = jnp.zeros((8, OUT_C), dtype=jnp.float32).at[0].set(bias)
    gamma2 = jnp.zeros((8, OUT_C), dtype=jnp.float32).at[0].set(gamma)
    beta2 = jnp.zeros((8, OUT_C), dtype=jnp.float32).at[0].set(beta)

    gather = _make_sc_gather()
    stage1 = pl.pallas_call(
        _stage1_kernel,
        grid=(NBLK,),
        compiler_params=pltpu.CompilerParams(
            vmem_limit_bytes=110 * 1024 * 1024),
        in_specs=[
            pl.BlockSpec((BLK, 16), lambda i: (i, 0)),
            pl.BlockSpec((BLK * K, 16), lambda i: (i, 0)),
            pl.BlockSpec((BLK * K, IN_C), lambda i: (i, 0)),
            pl.BlockSpec((16, 16), lambda i: (0, 0)),
            pl.BlockSpec((8, 16), lambda i: (0, 0)),
            pl.BlockSpec((16, P_PAD * IN_C), lambda i: (0, 0)),
            pl.BlockSpec((P_PAD * IN_C, OUT_C), lambda i: (0, 0)),
            pl.BlockSpec((8, OUT_C), lambda i: (0, 0)),
        ],
        out_specs=[
            pl.BlockSpec((BLK, OUT_C), lambda i: (i, 0)),
            pl.BlockSpec((8, 128), lambda i: (0, 0)),
        ],
        out_shape=[
            jax.ShapeDtypeStruct((QC, OUT_C), jnp.float32),
            jax.ShapeDtypeStruct((8, 128), jnp.float32),
        ],
    )

    convs, sums = [], []
    for h in range(CHUNKS):
        nb_h = lax.slice(nb1d, (h * ROWSC,), ((h + 1) * ROWSC,))
        fnb_h, snb_h = gather(f_flat, s_pad, nb_h)
        q_h = lax.slice(q16, (h * QC, 0), ((h + 1) * QC, 16))
        conv_h, sums_h = stage1(q_h, snb_h, fnb_h, kpt, kpn, expand, w2,
                                bias2)
        convs.append(conv_h)
        sums.append(sums_h)

    stage2 = pl.pallas_call(
        _stage2_kernel,
        grid=(NBLK,),
        in_specs=[
            pl.BlockSpec((BLK, OUT_C), lambda i: (i, 0)),
            pl.BlockSpec((8, 128), lambda i: (0, 0)),
            pl.BlockSpec((8, 128), lambda i: (0, 0)),
            pl.BlockSpec((8, OUT_C), lambda i: (0, 0)),
            pl.BlockSpec((8, OUT_C), lambda i: (0, 0)),
        ],
        out_specs=pl.BlockSpec((BLK, OUT_C), lambda i: (i, 0)),
        out_shape=jax.ShapeDtypeStruct((QC, OUT_C), jnp.float32),
    )
    outs = [stage2(c, sums[0], sums[1], gamma2, beta2) for c in convs]
    out = jnp.concatenate(outs, axis=0)

    return out.reshape(B, N, OUT_C)


# speedup vs baseline: 1.2095x; 1.0249x over previous
import functools

import jax
import jax.numpy as jnp
from jax import lax
from jax.experimental import pallas as pl
from jax.experimental.pallas import tpu as pltpu
from jax.experimental.pallas import tpu_sc as plsc

B = 2
N = 8192
M = 8192
K = 32
P = 15
P_PAD = 16
IN_C = 64
OUT_C = 64
SIGMA = 1.0
EPS = 1e-5

Q = B * N
ROWS = Q * K
NW = 32
CHUNKS = 4
QC = Q // CHUNKS
ROWSC = QC * K
RPW = ROWSC // NW
CH = 128
NCH = RPW // CH

BLK = 128
NBLK = QC // BLK


def _sc_gather_body(f_hbm, s_hbm, nb_hbm, fnb_hbm, snb_hbm,
                    idx_v, frows, srows, semf, sems):
    wid = lax.axis_index("s") * 2 + lax.axis_index("c")
    base0 = wid * RPW

    def body(j, carry):
        base = base0 + j * CH
        pltpu.sync_copy(nb_hbm.at[pl.ds(base, CH)], idx_v)
        cf = pltpu.async_copy(f_hbm.at[idx_v], frows, semf)
        cs = pltpu.async_copy(s_hbm.at[idx_v], srows, sems)
        cf.wait()
        cs.wait()
        pltpu.sync_copy(frows, fnb_hbm.at[pl.ds(base, CH)])
        pltpu.sync_copy(srows, snb_hbm.at[pl.ds(base, CH)])
        return carry

    lax.fori_loop(0, NCH, body, 0)


def _make_sc_gather():
    return pl.kernel(
        _sc_gather_body,
        out_type=[jax.ShapeDtypeStruct((ROWSC, IN_C), jnp.float32),
                  jax.ShapeDtypeStruct((ROWSC, 16), jnp.float32)],
        mesh=plsc.VectorSubcoreMesh(core_axis_name="c",
                                    subcore_axis_name="s"),
        scratch_types=[pltpu.VMEM((CH,), jnp.int32),
                       pltpu.VMEM((CH, IN_C), jnp.float32),
                       pltpu.VMEM((CH, 16), jnp.float32),
                       pltpu.SemaphoreType.DMA,
                       pltpu.SemaphoreType.DMA],
        compiler_params=pltpu.CompilerParams(use_tc_tiling_on_sc=False),
    )


def _stage1_kernel(q_ref, snb_ref, fnb_ref, kpt_ref, kpn_ref, exp_ref,
                   w2_ref, bias_ref, conv_ref, sums_ref):
    i = pl.program_id(0)

    qrep = jnp.broadcast_to(q_ref[...][:, None, :],
                            (BLK, K, 16)).reshape(BLK * K, 16)
    d3 = snb_ref[...] - qrep

    cross = jnp.dot(d3, kpt_ref[...],
                    preferred_element_type=jnp.float32)
    nrm = jnp.sum(d3 * d3, axis=1, keepdims=True)
    sq = jnp.maximum(nrm + kpn_ref[0][None, :] - 2.0 * cross, 0.0)
    infl = jnp.maximum(1.0 - jnp.sqrt(sq) / SIGMA, 0.0)

    inflx = jnp.dot(infl, exp_ref[...],
                    preferred_element_type=jnp.float32)
    fnb = fnb_ref[...]
    fnbx = jnp.concatenate([fnb] * P_PAD, axis=1)
    wf = inflx * fnbx
    weighted = jnp.sum(wf.reshape(BLK, K, P_PAD * IN_C), axis=1)

    conv = jnp.dot(weighted, w2_ref[...],
                   preferred_element_type=jnp.float32)

    nsum = jnp.sum(fnb, axis=1)
    valid = (jnp.abs(nsum) > 0.0).astype(jnp.float32).reshape(BLK, K)
    ncount = jnp.maximum(jnp.sum(valid, axis=1), 1.0)
    conv = conv / ncount[:, None] + bias_ref[0][None, :]

    conv_ref[...] = conv

    rows = jnp.stack([jnp.sum(conv, axis=0),
                      jnp.sum(conv * conv, axis=0)], axis=0)
    rows = jnp.concatenate(
        [rows, jnp.zeros((2, 128 - OUT_C), jnp.float32)], axis=1)
    s = jnp.concatenate([rows, jnp.zeros((6, 128), jnp.float32)], axis=0)

    @pl.when(i == 0)
    def _():
        sums_ref[...] = s

    @pl.when(i != 0)
    def _():
        sums_ref[...] += s


def _stage2_kernel(*refs):
    conv_ref = refs[0]
    sums_refs = refs[1:1 + CHUNKS]
    gamma_ref, beta_ref, out_ref = refs[1 + CHUNKS:]
    s0 = sums_refs[0][0, :OUT_C]
    s1 = sums_refs[0][1, :OUT_C]
    for sr in sums_refs[1:]:
        s0 = s0 + sr[0, :OUT_C]
        s1 = s1 + sr[1, :OUT_C]
    mean = s0 / float(Q)
    var = s1 / float(Q) - mean * mean
    inv = lax.rsqrt(var + EPS)
    xn = (conv_ref[...] - mean[None, :]) * inv[None, :]
    xn = xn * gamma_ref[0][None, :] + beta_ref[0][None, :]
    out_ref[...] = jnp.maximum(xn, 0.0)


def kernel(query, support, features, neighbors, kernel_points, weights,
           bias, gamma, beta):
    nb1d = (neighbors.astype(jnp.int32)
            + jnp.arange(B, dtype=jnp.int32)[:, None, None] * M
            ).reshape(ROWS)
    f_flat = features.reshape(B * M, IN_C)
    s_pad = jnp.concatenate(
        [support.reshape(B * M, 3),
         jnp.zeros((B * M, 13), jnp.float32)], axis=1)


    q16 = jnp.concatenate(
        [query.reshape(Q, 3), jnp.zeros((Q, 13), jnp.float32)], axis=1)

    kpt = jnp.zeros((16, 16), dtype=jnp.float32)
    kpt = kpt.at[:3, :P].set(kernel_points.T)
    kpn = jnp.zeros((8, 16), dtype=jnp.float32)
    kpn = kpn.at[0, :P].set(jnp.sum(kernel_points * kernel_points, axis=1))
    expand = jnp.zeros((16, P_PAD * IN_C), dtype=jnp.float32)
    for p in range(P):
        expand = expand.at[p, p * IN_C:(p + 1) * IN_C].set(1.0)

    w2 = jnp.concatenate(
        [weights.reshape(P * IN_C, OUT_C),
         jnp.zeros((IN_C, OUT_C), dtype=jnp.float32)], axis=0)

    bias2 = jnp.zeros((8, OUT_C), dtype=jnp.float32).at[0].set(bias)
    gamma2 = jnp.zeros((8, OUT_C), dtype=jnp.float32).at[0].set(gamma)
    beta2 = jnp.zeros((8, OUT_C), dtype=jnp.float32).at[0].set(beta)

    gather = _make_sc_gather()
    stage1 = pl.pallas_call(
        _stage1_kernel,
        grid=(NBLK,),
        compiler_params=pltpu.CompilerParams(
            vmem_limit_bytes=110 * 1024 * 1024),
        in_specs=[
            pl.BlockSpec((BLK, 16), lambda i: (i, 0)),
            pl.BlockSpec((BLK * K, 16), lambda i: (i, 0)),
            pl.BlockSpec((BLK * K, IN_C), lambda i: (i, 0)),
            pl.BlockSpec((16, 16), lambda i: (0, 0)),
            pl.BlockSpec((8, 16), lambda i: (0, 0)),
            pl.BlockSpec((16, P_PAD * IN_C), lambda i: (0, 0)),
            pl.BlockSpec((P_PAD * IN_C, OUT_C), lambda i: (0, 0)),
            pl.BlockSpec((8, OUT_C), lambda i: (0, 0)),
        ],
        out_specs=[
            pl.BlockSpec((BLK, OUT_C), lambda i: (i, 0)),
            pl.BlockSpec((8, 128), lambda i: (0, 0)),
        ],
        out_shape=[
            jax.ShapeDtypeStruct((QC, OUT_C), jnp.float32),
            jax.ShapeDtypeStruct((8, 128), jnp.float32),
        ],
    )

    convs, sums = [], []
    for h in range(CHUNKS):
        nb_h = lax.slice(nb1d, (h * ROWSC,), ((h + 1) * ROWSC,))
        fnb_h, snb_h = gather(f_flat, s_pad, nb_h)
        q_h = lax.slice(q16, (h * QC, 0), ((h + 1) * QC, 16))
        conv_h, sums_h = stage1(q_h, snb_h, fnb_h, kpt, kpn, expand, w2,
                                bias2)
        convs.append(conv_h)
        sums.append(sums_h)

    stage2 = pl.pallas_call(
        _stage2_kernel,
        grid=(NBLK,),
        in_specs=(
            [pl.BlockSpec((BLK, OUT_C), lambda i: (i, 0))]
            + [pl.BlockSpec((8, 128), lambda i: (0, 0))] * CHUNKS
            + [pl.BlockSpec((8, OUT_C), lambda i: (0, 0))] * 2
        ),
        out_specs=pl.BlockSpec((BLK, OUT_C), lambda i: (i, 0)),
        out_shape=jax.ShapeDtypeStruct((QC, OUT_C), jnp.float32),
    )
    outs = [stage2(c, *sums, gamma2, beta2) for c in convs]
    out = jnp.concatenate(outs, axis=0)

    return out.reshape(B, N, OUT_C)


# speedup vs baseline: 1.2466x; 1.0307x over previous
import functools

import jax
import jax.numpy as jnp
from jax import lax
from jax.experimental import pallas as pl
from jax.experimental.pallas import tpu as pltpu
from jax.experimental.pallas import tpu_sc as plsc

B = 2
N = 8192
M = 8192
K = 32
P = 15
P_PAD = 16
IN_C = 64
OUT_C = 64
SIGMA = 1.0
EPS = 1e-5

Q = B * N
ROWS = Q * K
NW = 32
CHUNKS = 4
QC = Q // CHUNKS
ROWSC = QC * K
RPW = ROWSC // NW
CH = 128
NCH = RPW // CH

BLK = 256
NBLK = QC // BLK


def _sc_gather_body(f_hbm, s_hbm, nb_hbm, fnb_hbm, snb_hbm,
                    idx_v, frows, srows, semf, sems):
    wid = lax.axis_index("s") * 2 + lax.axis_index("c")
    base0 = wid * RPW

    def body(j, carry):
        base = base0 + j * CH
        pltpu.sync_copy(nb_hbm.at[pl.ds(base, CH)], idx_v)
        cf = pltpu.async_copy(f_hbm.at[idx_v], frows, semf)
        cs = pltpu.async_copy(s_hbm.at[idx_v], srows, sems)
        cf.wait()
        cs.wait()
        pltpu.sync_copy(frows, fnb_hbm.at[pl.ds(base, CH)])
        pltpu.sync_copy(srows, snb_hbm.at[pl.ds(base, CH)])
        return carry

    lax.fori_loop(0, NCH, body, 0)


def _make_sc_gather():
    return pl.kernel(
        _sc_gather_body,
        out_type=[jax.ShapeDtypeStruct((ROWSC, IN_C), jnp.float32),
                  jax.ShapeDtypeStruct((ROWSC, 16), jnp.float32)],
        mesh=plsc.VectorSubcoreMesh(core_axis_name="c",
                                    subcore_axis_name="s"),
        scratch_types=[pltpu.VMEM((CH,), jnp.int32),
                       pltpu.VMEM((CH, IN_C), jnp.float32),
                       pltpu.VMEM((CH, 16), jnp.float32),
                       pltpu.SemaphoreType.DMA,
                       pltpu.SemaphoreType.DMA],
        compiler_params=pltpu.CompilerParams(use_tc_tiling_on_sc=False),
    )


def _stage1_kernel(q_ref, snb_ref, fnb_ref, kpt_ref, kpn_ref, exp_ref,
                   w2_ref, bias_ref, conv_ref, sums_ref):
    i = pl.program_id(0)

    qrep = jnp.broadcast_to(q_ref[...][:, None, :],
                            (BLK, K, 16)).reshape(BLK * K, 16)
    d3 = snb_ref[...] - qrep

    cross = jnp.dot(d3, kpt_ref[...],
                    preferred_element_type=jnp.float32)
    nrm = jnp.sum(d3 * d3, axis=1, keepdims=True)
    sq = jnp.maximum(nrm + kpn_ref[0][None, :] - 2.0 * cross, 0.0)
    infl = jnp.maximum(1.0 - jnp.sqrt(sq) / SIGMA, 0.0)

    inflx = jnp.dot(infl, exp_ref[...],
                    preferred_element_type=jnp.float32)
    fnb = fnb_ref[...]
    fnbx = jnp.concatenate([fnb] * P_PAD, axis=1)
    wf = inflx * fnbx
    weighted = jnp.sum(wf.reshape(BLK, K, P_PAD * IN_C), axis=1)

    conv = jnp.dot(weighted, w2_ref[...],
                   preferred_element_type=jnp.float32)

    nsum = jnp.sum(fnb, axis=1)
    valid = (jnp.abs(nsum) > 0.0).astype(jnp.float32).reshape(BLK, K)
    ncount = jnp.maximum(jnp.sum(valid, axis=1), 1.0)
    conv = conv / ncount[:, None] + bias_ref[0][None, :]

    conv_ref[...] = conv

    rows = jnp.stack([jnp.sum(conv, axis=0),
                      jnp.sum(conv * conv, axis=0)], axis=0)
    rows = jnp.concatenate(
        [rows, jnp.zeros((2, 128 - OUT_C), jnp.float32)], axis=1)
    s = jnp.concatenate([rows, jnp.zeros((6, 128), jnp.float32)], axis=0)

    @pl.when(i == 0)
    def _():
        sums_ref[...] = s

    @pl.when(i != 0)
    def _():
        sums_ref[...] += s


def _stage2_kernel(*refs):
    conv_ref = refs[0]
    sums_refs = refs[1:1 + CHUNKS]
    gamma_ref, beta_ref, out_ref = refs[1 + CHUNKS:]
    s0 = sums_refs[0][0, :OUT_C]
    s1 = sums_refs[0][1, :OUT_C]
    for sr in sums_refs[1:]:
        s0 = s0 + sr[0, :OUT_C]
        s1 = s1 + sr[1, :OUT_C]
    mean = s0 / float(Q)
    var = s1 / float(Q) - mean * mean
    inv = lax.rsqrt(var + EPS)
    xn = (conv_ref[...] - mean[None, :]) * inv[None, :]
    xn = xn * gamma_ref[0][None, :] + beta_ref[0][None, :]
    out_ref[...] = jnp.maximum(xn, 0.0)


def kernel(query, support, features, neighbors, kernel_points, weights,
           bias, gamma, beta):
    nb1d = (neighbors.astype(jnp.int32)
            + jnp.arange(B, dtype=jnp.int32)[:, None, None] * M
            ).reshape(ROWS)
    f_flat = features.reshape(B * M, IN_C)
    s_pad = jnp.concatenate(
        [support.reshape(B * M, 3),
         jnp.zeros((B * M, 13), jnp.float32)], axis=1)


    q16 = jnp.concatenate(
        [query.reshape(Q, 3), jnp.zeros((Q, 13), jnp.float32)], axis=1)

    kpt = jnp.zeros((16, 16), dtype=jnp.float32)
    kpt = kpt.at[:3, :P].set(kernel_points.T)
    kpn = jnp.zeros((8, 16), dtype=jnp.float32)
    kpn = kpn.at[0, :P].set(jnp.sum(kernel_points * kernel_points, axis=1))
    expand = jnp.zeros((16, P_PAD * IN_C), dtype=jnp.float32)
    for p in range(P):
        expand = expand.at[p, p * IN_C:(p + 1) * IN_C].set(1.0)

    w2 = jnp.concatenate(
        [weights.reshape(P * IN_C, OUT_C),
         jnp.zeros((IN_C, OUT_C), dtype=jnp.float32)], axis=0)

    bias2 = jnp.zeros((8, OUT_C), dtype=jnp.float32).at[0].set(bias)
    gamma2 = jnp.zeros((8, OUT_C), dtype=jnp.float32).at[0].set(gamma)
    beta2 = jnp.zeros((8, OUT_C), dtype=jnp.float32).at[0].set(beta)

    gather = _make_sc_gather()
    stage1 = pl.pallas_call(
        _stage1_kernel,
        grid=(NBLK,),
        compiler_params=pltpu.CompilerParams(
            vmem_limit_bytes=127 * 1024 * 1024),
        in_specs=[
            pl.BlockSpec((BLK, 16), lambda i: (i, 0)),
            pl.BlockSpec((BLK * K, 16), lambda i: (i, 0)),
            pl.BlockSpec((BLK * K, IN_C), lambda i: (i, 0)),
            pl.BlockSpec((16, 16), lambda i: (0, 0)),
            pl.BlockSpec((8, 16), lambda i: (0, 0)),
            pl.BlockSpec((16, P_PAD * IN_C), lambda i: (0, 0)),
            pl.BlockSpec((P_PAD * IN_C, OUT_C), lambda i: (0, 0)),
            pl.BlockSpec((8, OUT_C), lambda i: (0, 0)),
        ],
        out_specs=[
            pl.BlockSpec((BLK, OUT_C), lambda i: (i, 0)),
            pl.BlockSpec((8, 128), lambda i: (0, 0)),
        ],
        out_shape=[
            jax.ShapeDtypeStruct((QC, OUT_C), jnp.float32),
            jax.ShapeDtypeStruct((8, 128), jnp.float32),
        ],
    )

    convs, sums = [], []
    for h in range(CHUNKS):
        nb_h = lax.slice(nb1d, (h * ROWSC,), ((h + 1) * ROWSC,))
        fnb_h, snb_h = gather(f_flat, s_pad, nb_h)
        q_h = lax.slice(q16, (h * QC, 0), ((h + 1) * QC, 16))
        conv_h, sums_h = stage1(q_h, snb_h, fnb_h, kpt, kpn, expand, w2,
                                bias2)
        convs.append(conv_h)
        sums.append(sums_h)

    stage2 = pl.pallas_call(
        _stage2_kernel,
        grid=(NBLK,),
        in_specs=(
            [pl.BlockSpec((BLK, OUT_C), lambda i: (i, 0))]
            + [pl.BlockSpec((8, 128), lambda i: (0, 0))] * CHUNKS
            + [pl.BlockSpec((8, OUT_C), lambda i: (0, 0))] * 2
        ),
        out_specs=pl.BlockSpec((BLK, OUT_C), lambda i: (i, 0)),
        out_shape=jax.ShapeDtypeStruct((QC, OUT_C), jnp.float32),
    )
    outs = [stage2(c, *sums, gamma2, beta2) for c in convs]
    out = jnp.concatenate(outs, axis=0)

    return out.reshape(B, N, OUT_C)


# speedup vs baseline: 1.2591x; 1.0100x over previous
import functools

import jax
import jax.numpy as jnp
from jax import lax
from jax.experimental import pallas as pl
from jax.experimental.pallas import tpu as pltpu
from jax.experimental.pallas import tpu_sc as plsc

B = 2
N = 8192
M = 8192
K = 32
P = 15
P_PAD = 16
IN_C = 64
OUT_C = 64
SIGMA = 1.0
EPS = 1e-5

Q = B * N
ROWS = Q * K
NW = 32
CHUNKS = 8
QC = Q // CHUNKS
ROWSC = QC * K
RPW = ROWSC // NW
CH = 128
NCH = RPW // CH

BLK = 256
NBLK = QC // BLK


def _sc_gather_body(f_hbm, s_hbm, nb_hbm, fnb_hbm, snb_hbm,
                    idx_v, frows, srows, semf, sems):
    wid = lax.axis_index("s") * 2 + lax.axis_index("c")
    base0 = wid * RPW

    def body(j, carry):
        base = base0 + j * CH
        pltpu.sync_copy(nb_hbm.at[pl.ds(base, CH)], idx_v)
        cf = pltpu.async_copy(f_hbm.at[idx_v], frows, semf)
        cs = pltpu.async_copy(s_hbm.at[idx_v], srows, sems)
        cf.wait()
        cs.wait()
        pltpu.sync_copy(frows, fnb_hbm.at[pl.ds(base, CH)])
        pltpu.sync_copy(srows, snb_hbm.at[pl.ds(base, CH)])
        return carry

    lax.fori_loop(0, NCH, body, 0)


def _make_sc_gather():
    return pl.kernel(
        _sc_gather_body,
        out_type=[jax.ShapeDtypeStruct((ROWSC, IN_C), jnp.float32),
                  jax.ShapeDtypeStruct((ROWSC, 16), jnp.float32)],
        mesh=plsc.VectorSubcoreMesh(core_axis_name="c",
                                    subcore_axis_name="s"),
        scratch_types=[pltpu.VMEM((CH,), jnp.int32),
                       pltpu.VMEM((CH, IN_C), jnp.float32),
                       pltpu.VMEM((CH, 16), jnp.float32),
                       pltpu.SemaphoreType.DMA,
                       pltpu.SemaphoreType.DMA],
        compiler_params=pltpu.CompilerParams(use_tc_tiling_on_sc=False),
    )


def _stage1_kernel(q_ref, snb_ref, fnb_ref, kpt_ref, kpn_ref, exp_ref,
                   w2_ref, bias_ref, conv_ref, sums_ref):
    i = pl.program_id(0)

    qrep = jnp.broadcast_to(q_ref[...][:, None, :],
                            (BLK, K, 16)).reshape(BLK * K, 16)
    d3 = snb_ref[...] - qrep

    cross = jnp.dot(d3, kpt_ref[...],
                    preferred_element_type=jnp.float32)
    nrm = jnp.sum(d3 * d3, axis=1, keepdims=True)
    sq = jnp.maximum(nrm + kpn_ref[0][None, :] - 2.0 * cross, 0.0)
    infl = jnp.maximum(1.0 - jnp.sqrt(sq) / SIGMA, 0.0)

    inflx = jnp.dot(infl, exp_ref[...],
                    preferred_element_type=jnp.float32)
    fnb = fnb_ref[...]
    fnbx = jnp.concatenate([fnb] * P_PAD, axis=1)
    wf = inflx * fnbx
    weighted = jnp.sum(wf.reshape(BLK, K, P_PAD * IN_C), axis=1)

    conv = jnp.dot(weighted, w2_ref[...],
                   preferred_element_type=jnp.float32)

    nsum = jnp.sum(fnb, axis=1)
    valid = (jnp.abs(nsum) > 0.0).astype(jnp.float32).reshape(BLK, K)
    ncount = jnp.maximum(jnp.sum(valid, axis=1), 1.0)
    conv = conv / ncount[:, None] + bias_ref[0][None, :]

    conv_ref[...] = conv

    rows = jnp.stack([jnp.sum(conv, axis=0),
                      jnp.sum(conv * conv, axis=0)], axis=0)
    rows = jnp.concatenate(
        [rows, jnp.zeros((2, 128 - OUT_C), jnp.float32)], axis=1)
    s = jnp.concatenate([rows, jnp.zeros((6, 128), jnp.float32)], axis=0)

    @pl.when(i == 0)
    def _():
        sums_ref[...] = s

    @pl.when(i != 0)
    def _():
        sums_ref[...] += s


def _stage2_kernel(*refs):
    conv_ref = refs[0]
    sums_refs = refs[1:1 + CHUNKS]
    gamma_ref, beta_ref, out_ref = refs[1 + CHUNKS:]
    s0 = sums_refs[0][0, :OUT_C]
    s1 = sums_refs[0][1, :OUT_C]
    for sr in sums_refs[1:]:
        s0 = s0 + sr[0, :OUT_C]
        s1 = s1 + sr[1, :OUT_C]
    mean = s0 / float(Q)
    var = s1 / float(Q) - mean * mean
    inv = lax.rsqrt(var + EPS)
    xn = (conv_ref[...] - mean[None, :]) * inv[None, :]
    xn = xn * gamma_ref[0][None, :] + beta_ref[0][None, :]
    out_ref[...] = jnp.maximum(xn, 0.0)


def kernel(query, support, features, neighbors, kernel_points, weights,
           bias, gamma, beta):
    nb1d = (neighbors.astype(jnp.int32)
            + jnp.arange(B, dtype=jnp.int32)[:, None, None] * M
            ).reshape(ROWS)
    f_flat = features.reshape(B * M, IN_C)
    s_pad = jnp.concatenate(
        [support.reshape(B * M, 3),
         jnp.zeros((B * M, 13), jnp.float32)], axis=1)


    q16 = jnp.concatenate(
        [query.reshape(Q, 3), jnp.zeros((Q, 13), jnp.float32)], axis=1)

    kpt = jnp.zeros((16, 16), dtype=jnp.float32)
    kpt = kpt.at[:3, :P].set(kernel_points.T)
    kpn = jnp.zeros((8, 16), dtype=jnp.float32)
    kpn = kpn.at[0, :P].set(jnp.sum(kernel_points * kernel_points, axis=1))
    expand = jnp.zeros((16, P_PAD * IN_C), dtype=jnp.float32)
    for p in range(P):
        expand = expand.at[p, p * IN_C:(p + 1) * IN_C].set(1.0)

    w2 = jnp.concatenate(
        [weights.reshape(P * IN_C, OUT_C),
         jnp.zeros((IN_C, OUT_C), dtype=jnp.float32)], axis=0)

    bias2 = jnp.zeros((8, OUT_C), dtype=jnp.float32).at[0].set(bias)
    gamma2 = jnp.zeros((8, OUT_C), dtype=jnp.float32).at[0].set(gamma)
    beta2 = jnp.zeros((8, OUT_C), dtype=jnp.float32).at[0].set(beta)

    gather = _make_sc_gather()
    stage1 = pl.pallas_call(
        _stage1_kernel,
        grid=(NBLK,),
        compiler_params=pltpu.CompilerParams(
            vmem_limit_bytes=127 * 1024 * 1024),
        in_specs=[
            pl.BlockSpec((BLK, 16), lambda i: (i, 0)),
            pl.BlockSpec((BLK * K, 16), lambda i: (i, 0)),
            pl.BlockSpec((BLK * K, IN_C), lambda i: (i, 0)),
            pl.BlockSpec((16, 16), lambda i: (0, 0)),
            pl.BlockSpec((8, 16), lambda i: (0, 0)),
            pl.BlockSpec((16, P_PAD * IN_C), lambda i: (0, 0)),
            pl.BlockSpec((P_PAD * IN_C, OUT_C), lambda i: (0, 0)),
            pl.BlockSpec((8, OUT_C), lambda i: (0, 0)),
        ],
        out_specs=[
            pl.BlockSpec((BLK, OUT_C), lambda i: (i, 0)),
            pl.BlockSpec((8, 128), lambda i: (0, 0)),
        ],
        out_shape=[
            jax.ShapeDtypeStruct((QC, OUT_C), jnp.float32),
            jax.ShapeDtypeStruct((8, 128), jnp.float32),
        ],
    )

    convs, sums = [], []
    for h in range(CHUNKS):
        nb_h = lax.slice(nb1d, (h * ROWSC,), ((h + 1) * ROWSC,))
        fnb_h, snb_h = gather(f_flat, s_pad, nb_h)
        q_h = lax.slice(q16, (h * QC, 0), ((h + 1) * QC, 16))
        conv_h, sums_h = stage1(q_h, snb_h, fnb_h, kpt, kpn, expand, w2,
                                bias2)
        convs.append(conv_h)
        sums.append(sums_h)

    stage2 = pl.pallas_call(
        _stage2_kernel,
        grid=(NBLK,),
        in_specs=(
            [pl.BlockSpec((BLK, OUT_C), lambda i: (i, 0))]
            + [pl.BlockSpec((8, 128), lambda i: (0, 0))] * CHUNKS
            + [pl.BlockSpec((8, OUT_C), lambda i: (0, 0))] * 2
        ),
        out_specs=pl.BlockSpec((BLK, OUT_C), lambda i: (i, 0)),
        out_shape=jax.ShapeDtypeStruct((QC, OUT_C), jnp.float32),
    )
    outs = [stage2(c, *sums, gamma2, beta2) for c in convs]
    out = jnp.concatenate(outs, axis=0)

    return out.reshape(B, N, OUT_C)
